# Initial kernel scaffold; baseline (speedup 1.0000x reference)
import jax
import jax.numpy as jnp
from jax import lax
from jax.experimental import pallas as pl
from jax.experimental.pallas import tpu as pltpu
from jax.experimental.pallas import tpu_sc as plsc

N = 10000
E = 160000
D_IN = 256
D_HID = 512
D_OUT = 256

NC = 2
NS = 16
BA = 128
SINK = 16
RPT = N // NS
ZR = 125

EPT_A = E // NS
NBA = 80
PAD_A = NBA * BA - EPT_A

EPT_D = E // (NC * NS)
NBD = 40
PAD_D = NBD * BA - EPT_D

_MESH = dict(core_axis_name="c", subcore_axis_name="s", num_cores=NC,
             num_subcores=NS)


def _deg_body(col_hbm, deg_hbm, colbuf, ones_msg, zbuf, accum, sem):
    c = lax.axis_index("c")
    s = lax.axis_index("s")
    zf = jnp.zeros((16,), jnp.float32)
    of = jnp.ones((16,), jnp.float32)

    def fill(i, _):
        zbuf[i, :] = zf
        return 0
    lax.fori_loop(0, ZR, fill, 0)

    def fill1(i, _):
        ones_msg[i, :] = of
        return 0
    lax.fori_loop(0, BA, fill1, 0)

    pltpu.sync_copy(col_hbm.at[pl.ds((c * NS + s) * NBD, NBD)], colbuf)
    for z in range(RPT // ZR):
        pltpu.sync_copy(zbuf, accum.at[pl.ds(s * RPT + z * ZR, ZR)])
    plsc.subcore_barrier()

    def batch(b, _):
        pltpu.sync_copy(ones_msg, accum.at[colbuf.at[b]], add=True)
        return 0
    lax.fori_loop(0, NBD, batch, 0)
    plsc.subcore_barrier()
    pltpu.sync_copy(accum.at[pl.ds(s * RPT, RPT)],
                    deg_hbm.at[pl.ds(c * N + s * RPT, RPT)])


_deg_call = pl.kernel(
    _deg_body,
    out_type=jax.ShapeDtypeStruct((NC * N, 16), jnp.float32),
    mesh=plsc.VectorSubcoreMesh(**_MESH),
    scratch_types=[
        pltpu.VMEM((NBD, BA), jnp.int32),
        pltpu.VMEM((BA, 16), jnp.float32),
        pltpu.VMEM((ZR, 16), jnp.float32),
        pltpu.VMEM_SHARED((N + SINK, 16), jnp.float32),
        pltpu.SemaphoreType.DMA,
    ],
)


def _make_agg(C):
    CC = C // NC

    def body(u_hbm, rowg_hbm, col_hbm, out_hbm,
             rowbuf, colbuf, msg, zbuf, accum, sem):
        c = lax.axis_index("c")
        s = lax.axis_index("s")
        zf = jnp.zeros((16,), jnp.float32)

        def fill(i, _):
            zbuf[i // 8, pl.ds((i % 8) * 16, 16)] = zf
            return 0
        lax.fori_loop(0, ZR * 8, fill, 0)

        pltpu.sync_copy(col_hbm.at[pl.ds(s * NBA, NBA)], colbuf)
        for k in range(CC):
            chunk = c * CC + k
            for z in range(RPT // ZR):
                pltpu.sync_copy(zbuf, accum.at[pl.ds(s * RPT + z * ZR, ZR)])
            pltpu.sync_copy(
                rowg_hbm.at[pl.ds((chunk * NS + s) * NBA, NBA)], rowbuf)
            plsc.subcore_barrier()

            def group(g, _):
                cps = [
                    pltpu.async_copy(u_hbm.at[rowbuf.at[g * 4 + j]],
                                     msg.at[j], sem)
                    for j in range(4)
                ]
                for cp in cps:
                    cp.wait()
                for j in range(4):
                    pltpu.sync_copy(msg.at[j], accum.at[colbuf.at[g * 4 + j]],
                                    add=True)
                return 0
            lax.fori_loop(0, NBA // 4, group, 0)
            plsc.subcore_barrier()
            pltpu.sync_copy(accum.at[pl.ds(s * RPT, RPT)],
                            out_hbm.at[pl.ds(chunk * N + s * RPT, RPT)])
            plsc.subcore_barrier()

    return pl.kernel(
        body,
        out_type=jax.ShapeDtypeStruct((C * N, 128), jnp.float32),
        mesh=plsc.VectorSubcoreMesh(**_MESH),
        scratch_types=[
            pltpu.VMEM((NBA, BA), jnp.int32),
            pltpu.VMEM((NBA, BA), jnp.int32),
            pltpu.VMEM((4, BA, 128), jnp.float32),
            pltpu.VMEM((ZR, 128), jnp.float32),
            pltpu.VMEM_SHARED((N + SINK, 128), jnp.float32),
            pltpu.SemaphoreType.DMA,
        ],
    )


_agg4_call = _make_agg(4)
_agg2_call = _make_agg(2)


BR = 400
GRID = N // BR


def _dinv_from(deg_ref):
    deg = deg_ref[0, :, 0] + deg_ref[1, :, 0] + 1.0
    return lax.rsqrt(deg)[:, None]


def _tc1_body(x_ref, w_ref, deg_ref, u_ref):
    dinv = _dinv_from(deg_ref)
    u = jnp.dot(x_ref[...], w_ref[...],
                preferred_element_type=jnp.float32) * dinv
    for k in range(4):
        u_ref[k] = u[:, k * 128:(k + 1) * 128]


_tc1_call = pl.pallas_call(
    _tc1_body,
    grid=(GRID,),
    in_specs=[
        pl.BlockSpec((BR, D_IN), lambda i: (i, 0)),
        pl.BlockSpec((D_IN, D_HID), lambda i: (0, 0)),
        pl.BlockSpec((2, BR, 16), lambda i: (0, i, 0)),
    ],
    out_specs=pl.BlockSpec((4, BR, 128), lambda i: (0, i, 0)),
    out_shape=jax.ShapeDtypeStruct((4, N, 128), jnp.float32),
)


def _tc2_body(agg_ref, u_ref, deg_ref, b1_ref, w2_ref, v_ref):
    dinv = _dinv_from(deg_ref)
    v = jnp.zeros((BR, D_OUT), jnp.float32)
    for k in range(4):
        h = jnp.maximum(
            (agg_ref[k] + u_ref[k]) * dinv + b1_ref[0, k * 128:(k + 1) * 128],
            0.0)
        v = v + jnp.dot(h, w2_ref[k * 128:(k + 1) * 128, :],
                        preferred_element_type=jnp.float32)
    v = v * dinv
    for k in range(2):
        v_ref[k] = v[:, k * 128:(k + 1) * 128]


_tc2_call = pl.pallas_call(
    _tc2_body,
    grid=(GRID,),
    in_specs=[
        pl.BlockSpec((4, BR, 128), lambda i: (0, i, 0)),
        pl.BlockSpec((4, BR, 128), lambda i: (0, i, 0)),
        pl.BlockSpec((2, BR, 16), lambda i: (0, i, 0)),
        pl.BlockSpec((1, D_HID), lambda i: (0, 0)),
        pl.BlockSpec((D_HID, D_OUT), lambda i: (0, 0)),
    ],
    out_specs=pl.BlockSpec((2, BR, 128), lambda i: (0, i, 0)),
    out_shape=jax.ShapeDtypeStruct((2, N, 128), jnp.float32),
)


def _tc3_body(agg_ref, v_ref, deg_ref, b2_ref, o_ref):
    dinv = _dinv_from(deg_ref)
    for k in range(2):
        o_ref[:, k * 128:(k + 1) * 128] = jnp.maximum(
            (agg_ref[k] + v_ref[k]) * dinv + b2_ref[0, k * 128:(k + 1) * 128],
            0.0)


_tc3_call = pl.pallas_call(
    _tc3_body,
    grid=(GRID,),
    in_specs=[
        pl.BlockSpec((2, BR, 128), lambda i: (0, i, 0)),
        pl.BlockSpec((2, BR, 128), lambda i: (0, i, 0)),
        pl.BlockSpec((2, BR, 16), lambda i: (0, i, 0)),
        pl.BlockSpec((1, D_OUT), lambda i: (0, 0)),
    ],
    out_specs=pl.BlockSpec((BR, D_OUT), lambda i: (i, 0)),
    out_shape=jax.ShapeDtypeStruct((N, D_OUT), jnp.float32),
)


def _prep_indices(edge_index):
    row = edge_index[0].astype(jnp.int32)
    col = edge_index[1].astype(jnp.int32)

    row_t = row.reshape(NS, EPT_A)
    col_t = col.reshape(NS, EPT_A)
    prange = jnp.arange(PAD_A, dtype=jnp.int32)
    pad_row = (prange * 997) % N
    pad_col = N + (prange % SINK)
    row_tp = jnp.concatenate(
        [row_t, jnp.broadcast_to(pad_row, (NS, PAD_A))], axis=1)
    col_tp = jnp.concatenate(
        [col_t, jnp.broadcast_to(pad_col, (NS, PAD_A))], axis=1)
    col_a = col_tp.reshape(NS * NBA, BA)
    offs = (jnp.arange(4, dtype=jnp.int32) * N)[:, None, None]
    rowg4 = (row_tp[None] + offs).reshape(4 * NS * NBA, BA)
    rowg2 = (row_tp[None] + offs[:2]).reshape(2 * NS * NBA, BA)

    col_d = col.reshape(NC * NS, EPT_D)
    drange = jnp.arange(PAD_D, dtype=jnp.int32)
    dpad = N + (drange % SINK)
    col_dp = jnp.concatenate(
        [col_d, jnp.broadcast_to(dpad, (NC * NS, PAD_D))], axis=1)
    col_deg = col_dp.reshape(NC * NS * NBD, BA)
    return rowg4, rowg2, col_a, col_deg


def kernel(x, edge_index, W1, b1, W2, b2):
    rowg4, rowg2, col_a, col_deg = _prep_indices(edge_index)

    deg2 = _deg_call(col_deg).reshape(NC, N, 16)
    u = _tc1_call(x, W1, deg2)
    agg1 = _agg4_call(u.reshape(4 * N, 128), rowg4, col_a).reshape(4, N, 128)
    v = _tc2_call(agg1, u, deg2, b1.reshape(1, D_HID), W2)
    agg2 = _agg2_call(v.reshape(2 * N, 128), rowg2, col_a).reshape(2, N, 128)
    return _tc3_call(agg2, v, deg2, b2.reshape(1, D_OUT))


# speedup vs baseline: 10.3220x; 10.3220x over previous
import functools

import jax
import jax.numpy as jnp
from jax import lax
from jax.experimental import pallas as pl
from jax.experimental.pallas import tpu as pltpu
from jax.experimental.pallas import tpu_sc as plsc

N = 10000
E = 160000
D_IN = 256
D_HID = 512
D_OUT = 256

NC = 2
NS = 16
BA = 128
CH = 64
C1 = D_HID // CH
C2 = D_OUT // CH
SINK = 16
RPT = N // NS
ZR = 125

EPT_A = E // NS
NBA = 80
PAD_A = NBA * BA - EPT_A

EPT_D = E // (NC * NS)
NBD = 40
PAD_D = NBD * BA - EPT_D

_MESH = dict(core_axis_name="c", subcore_axis_name="s", num_cores=NC,
             num_subcores=NS)


def _deg_body(col_hbm, deg_hbm, colbuf, ones_msg, zbuf, accum, sem):
    c = lax.axis_index("c")
    s = lax.axis_index("s")
    zf = jnp.zeros((16,), jnp.float32)
    of = jnp.ones((16,), jnp.float32)

    def fill(i, _):
        zbuf[i, :] = zf
        return 0
    lax.fori_loop(0, ZR, fill, 0)

    def fill1(i, _):
        ones_msg[i, :] = of
        return 0
    lax.fori_loop(0, BA, fill1, 0)

    pltpu.sync_copy(col_hbm.at[pl.ds((c * NS + s) * NBD, NBD)], colbuf)
    for z in range(RPT // ZR):
        pltpu.sync_copy(zbuf, accum.at[pl.ds(s * RPT + z * ZR, ZR)])
    plsc.subcore_barrier()

    def batch(b, _):
        pltpu.sync_copy(ones_msg, accum.at[colbuf.at[b]], add=True)
        return 0
    lax.fori_loop(0, NBD, batch, 0)
    plsc.subcore_barrier()
    pltpu.sync_copy(accum.at[pl.ds(s * RPT, RPT)],
                    deg_hbm.at[pl.ds(c * N + s * RPT, RPT)])


@functools.cache
def _deg_call():
  return pl.kernel(
    _deg_body,
    out_type=jax.ShapeDtypeStruct((NC * N, 16), jnp.float32),
    mesh=plsc.VectorSubcoreMesh(**_MESH),
    compiler_params=pltpu.CompilerParams(use_tc_tiling_on_sc=False),
    scratch_types=[
        pltpu.VMEM((NBD, BA), jnp.int32),
        pltpu.VMEM((BA, 16), jnp.float32),
        pltpu.VMEM((ZR, 16), jnp.float32),
        pltpu.VMEM_SHARED((N + SINK, 16), jnp.float32),
        pltpu.SemaphoreType.DMA,
    ],
  )


@functools.cache
def _make_agg(C):
    CC = C // NC

    def body(u_hbm, rowg_hbm, col_hbm, out_hbm,
             rowbuf, colbuf, msg, zbuf, accum, sem):
        c = lax.axis_index("c")
        s = lax.axis_index("s")
        zf = jnp.zeros((16,), jnp.float32)

        nzc = CH // 16
        def fill(i, _):
            zbuf[i // nzc, pl.ds((i % nzc) * 16, 16)] = zf
            return 0
        lax.fori_loop(0, ZR * nzc, fill, 0)

        pltpu.sync_copy(col_hbm.at[pl.ds(s * NBA, NBA)], colbuf)
        for k in range(CC):
            chunk = c * CC + k
            for z in range(RPT // ZR):
                pltpu.sync_copy(zbuf, accum.at[pl.ds(s * RPT + z * ZR, ZR)])
            pltpu.sync_copy(
                rowg_hbm.at[pl.ds((chunk * NS + s) * NBA, NBA)], rowbuf)
            plsc.subcore_barrier()

            def group(g, _):
                cps = [
                    pltpu.async_copy(u_hbm.at[rowbuf.at[g * 4 + j]],
                                     msg.at[j], sem)
                    for j in range(4)
                ]
                for cp in cps:
                    cp.wait()
                for j in range(4):
                    pltpu.sync_copy(msg.at[j], accum.at[colbuf.at[g * 4 + j]],
                                    add=True)
                return 0
            lax.fori_loop(0, NBA // 4, group, 0)
            plsc.subcore_barrier()
            pltpu.sync_copy(accum.at[pl.ds(s * RPT, RPT)],
                            out_hbm.at[pl.ds(chunk * N + s * RPT, RPT)])
            plsc.subcore_barrier()

    return pl.kernel(
        body,
        out_type=jax.ShapeDtypeStruct((C * N, CH), jnp.float32),
        mesh=plsc.VectorSubcoreMesh(**_MESH),
        compiler_params=pltpu.CompilerParams(use_tc_tiling_on_sc=False),
        scratch_types=[
            pltpu.VMEM((NBA, BA), jnp.int32),
            pltpu.VMEM((NBA, BA), jnp.int32),
            pltpu.VMEM((4, BA, CH), jnp.float32),
            pltpu.VMEM((ZR, CH), jnp.float32),
            pltpu.VMEM_SHARED((N + SINK, CH), jnp.float32),
            pltpu.SemaphoreType.DMA,
        ],
    )


BR = 400
GRID = N // BR


def _dinv_from(deg_ref):
    deg = deg_ref[0, :, 0] + deg_ref[1, :, 0] + 1.0
    return lax.rsqrt(deg)[:, None]


def _tc1_body(x_ref, w_ref, deg_ref, u_ref):
    dinv = _dinv_from(deg_ref)
    u = jnp.dot(x_ref[...], w_ref[...],
                preferred_element_type=jnp.float32) * dinv
    for k in range(C1):
        u_ref[k] = u[:, k * CH:(k + 1) * CH]


_tc1_call = pl.pallas_call(
    _tc1_body,
    grid=(GRID,),
    in_specs=[
        pl.BlockSpec((BR, D_IN), lambda i: (i, 0)),
        pl.BlockSpec((D_IN, D_HID), lambda i: (0, 0)),
        pl.BlockSpec((2, BR, 16), lambda i: (0, i, 0)),
    ],
    out_specs=pl.BlockSpec((C1, BR, CH), lambda i: (0, i, 0)),
    out_shape=jax.ShapeDtypeStruct((C1, N, CH), jnp.float32),
)


def _tc2_body(agg_ref, u_ref, deg_ref, b1_ref, w2_ref, v_ref):
    dinv = _dinv_from(deg_ref)
    pre = jnp.concatenate(
        [agg_ref[k] + u_ref[k] for k in range(C1)], axis=1)
    h = jnp.maximum(pre * dinv + b1_ref[0], 0.0)
    v = jnp.dot(h, w2_ref[...], preferred_element_type=jnp.float32) * dinv
    for k in range(C2):
        v_ref[k] = v[:, k * CH:(k + 1) * CH]


_tc2_call = pl.pallas_call(
    _tc2_body,
    grid=(GRID,),
    in_specs=[
        pl.BlockSpec((C1, BR, CH), lambda i: (0, i, 0)),
        pl.BlockSpec((C1, BR, CH), lambda i: (0, i, 0)),
        pl.BlockSpec((2, BR, 16), lambda i: (0, i, 0)),
        pl.BlockSpec((1, D_HID), lambda i: (0, 0)),
        pl.BlockSpec((D_HID, D_OUT), lambda i: (0, 0)),
    ],
    out_specs=pl.BlockSpec((C2, BR, CH), lambda i: (0, i, 0)),
    out_shape=jax.ShapeDtypeStruct((C2, N, CH), jnp.float32),
)


def _tc3_body(agg_ref, v_ref, deg_ref, b2_ref, o_ref):
    dinv = _dinv_from(deg_ref)
    for k in range(C2):
        o_ref[:, k * CH:(k + 1) * CH] = jnp.maximum(
            (agg_ref[k] + v_ref[k]) * dinv + b2_ref[0, k * CH:(k + 1) * CH],
            0.0)


_tc3_call = pl.pallas_call(
    _tc3_body,
    grid=(GRID,),
    in_specs=[
        pl.BlockSpec((C2, BR, CH), lambda i: (0, i, 0)),
        pl.BlockSpec((C2, BR, CH), lambda i: (0, i, 0)),
        pl.BlockSpec((2, BR, 16), lambda i: (0, i, 0)),
        pl.BlockSpec((1, D_OUT), lambda i: (0, 0)),
    ],
    out_specs=pl.BlockSpec((BR, D_OUT), lambda i: (i, 0)),
    out_shape=jax.ShapeDtypeStruct((N, D_OUT), jnp.float32),
)


def _prep_indices(edge_index):
    row = edge_index[0].astype(jnp.int32)
    col = edge_index[1].astype(jnp.int32)

    row_t = row.reshape(NS, EPT_A)
    col_t = col.reshape(NS, EPT_A)
    prange = jnp.arange(PAD_A, dtype=jnp.int32)
    pad_row = (prange * 997) % N
    pad_col = N + (prange % SINK)
    row_tp = jnp.concatenate(
        [row_t, jnp.broadcast_to(pad_row, (NS, PAD_A))], axis=1)
    col_tp = jnp.concatenate(
        [col_t, jnp.broadcast_to(pad_col, (NS, PAD_A))], axis=1)
    col_a = col_tp.reshape(NS * NBA, BA)
    offs = (jnp.arange(C1, dtype=jnp.int32) * N)[:, None, None]
    rowg1 = (row_tp[None] + offs).reshape(C1 * NS * NBA, BA)
    rowg2 = (row_tp[None] + offs[:C2]).reshape(C2 * NS * NBA, BA)

    col_d = col.reshape(NC * NS, EPT_D)
    drange = jnp.arange(PAD_D, dtype=jnp.int32)
    dpad = N + (drange % SINK)
    col_dp = jnp.concatenate(
        [col_d, jnp.broadcast_to(dpad, (NC * NS, PAD_D))], axis=1)
    col_deg = col_dp.reshape(NC * NS * NBD, BA)
    return rowg1, rowg2, col_a, col_deg


def kernel(x, edge_index, W1, b1, W2, b2):
    rowg1, rowg2, col_a, col_deg = _prep_indices(edge_index)

    deg2 = _deg_call()(col_deg).reshape(NC, N, 16)
    u = _tc1_call(x, W1, deg2)
    agg1 = _make_agg(C1)(u.reshape(C1 * N, CH), rowg1,
                         col_a).reshape(C1, N, CH)
    v = _tc2_call(agg1, u, deg2, b1.reshape(1, D_HID), W2)
    agg2 = _make_agg(C2)(v.reshape(C2 * N, CH), rowg2,
                         col_a).reshape(C2, N, CH)
    return _tc3_call(agg2, v, deg2, b2.reshape(1, D_OUT))


# speedup vs baseline: 12.8948x; 1.2493x over previous
import functools

import jax
import jax.numpy as jnp
from jax import lax
from jax.experimental import pallas as pl
from jax.experimental.pallas import tpu as pltpu
from jax.experimental.pallas import tpu_sc as plsc

N = 10000
E = 160000
D_IN = 256
D_HID = 512
D_OUT = 256

NC = 2
NS = 16
BA = 128
CH = 64
C1 = D_HID // CH
C2 = D_OUT // CH
SINK = 16
RPT = N // NS
ZR = 25

EPT_A = E // NS
NBA = 80
PAD_A = NBA * BA - EPT_A

EPT_D = E // (NC * NS)
NBD = 40
PAD_D = NBD * BA - EPT_D

_MESH = dict(core_axis_name="c", subcore_axis_name="s", num_cores=NC,
             num_subcores=NS)


def _deg_body(col_hbm, deg_hbm, colbuf, ones_msg, zbuf, accum, sem):
    c = lax.axis_index("c")
    s = lax.axis_index("s")
    zf = jnp.zeros((16,), jnp.float32)
    of = jnp.ones((16,), jnp.float32)

    def fill(i, _):
        zbuf[i, :] = zf
        return 0
    lax.fori_loop(0, ZR, fill, 0)

    def fill1(i, _):
        ones_msg[i, :] = of
        return 0
    lax.fori_loop(0, BA, fill1, 0)

    pltpu.sync_copy(col_hbm.at[pl.ds((c * NS + s) * NBD, NBD)], colbuf)
    for z in range(RPT // ZR):
        pltpu.sync_copy(zbuf, accum.at[pl.ds(s * RPT + z * ZR, ZR)])
    plsc.subcore_barrier()

    def batch(b, _):
        pltpu.sync_copy(ones_msg, accum.at[colbuf.at[b]], add=True)
        return 0
    lax.fori_loop(0, NBD, batch, 0)
    plsc.subcore_barrier()
    pltpu.sync_copy(accum.at[pl.ds(s * RPT, RPT)],
                    deg_hbm.at[pl.ds(c * N + s * RPT, RPT)])


@functools.cache
def _deg_call():
  return pl.kernel(
    _deg_body,
    out_type=jax.ShapeDtypeStruct((NC * N, 16), jnp.float32),
    mesh=plsc.VectorSubcoreMesh(**_MESH),
    compiler_params=pltpu.CompilerParams(use_tc_tiling_on_sc=False),
    scratch_types=[
        pltpu.VMEM((NBD, BA), jnp.int32),
        pltpu.VMEM((BA, 16), jnp.float32),
        pltpu.VMEM((ZR, 16), jnp.float32),
        pltpu.VMEM_SHARED((N + SINK, 16), jnp.float32),
        pltpu.SemaphoreType.DMA,
    ],
  )


@functools.cache
def _make_agg(C):
    CC = C // NC

    NG = NBA // 4

    def body(u_hbm, rowg_hbm, col_hbm, out_hbm,
             rowbuf, colbuf, msga, msgb, zbuf, accum, sema, semb):
        c = lax.axis_index("c")
        s = lax.axis_index("s")
        zf = jnp.zeros((16,), jnp.float32)

        nzc = CH // 16
        def fill(i, _):
            zbuf[i // nzc, pl.ds((i % nzc) * 16, 16)] = zf
            return 0
        lax.fori_loop(0, ZR * nzc, fill, 0)

        def fire(buf, sem, g):
            for j in range(4):
                pltpu.async_copy(u_hbm.at[rowbuf.at[g * 4 + j]],
                                 buf.at[j], sem)

        def drain(buf, sem):
            for j in range(4):
                pltpu.make_async_copy(u_hbm.at[rowbuf.at[j]],
                                      buf.at[j], sem).wait()

        def scat(buf, g):
            for j in range(4):
                pltpu.sync_copy(buf.at[j], accum.at[colbuf.at[g * 4 + j]],
                                add=True)

        pltpu.sync_copy(col_hbm.at[pl.ds(s * NBA, NBA)], colbuf)
        for k in range(CC):
            chunk = c * CC + k
            for z in range(RPT // ZR):
                pltpu.sync_copy(zbuf, accum.at[pl.ds(s * RPT + z * ZR, ZR)])
            pltpu.sync_copy(
                rowg_hbm.at[pl.ds((chunk * NS + s) * NBA, NBA)], rowbuf)
            plsc.subcore_barrier()

            fire(msga, sema, 0)
            fire(msgb, semb, 1)

            def pipe(i, _):
                g0 = 2 * i
                drain(msga, sema)
                scat(msga, g0)
                fire(msga, sema, g0 + 2)
                drain(msgb, semb)
                scat(msgb, g0 + 1)
                fire(msgb, semb, g0 + 3)
                return 0
            lax.fori_loop(0, NG // 2 - 1, pipe, 0)
            drain(msga, sema)
            scat(msga, NG - 2)
            drain(msgb, semb)
            scat(msgb, NG - 1)

            plsc.subcore_barrier()
            pltpu.sync_copy(accum.at[pl.ds(s * RPT, RPT)],
                            out_hbm.at[pl.ds(chunk * N + s * RPT, RPT)])
            plsc.subcore_barrier()

    return pl.kernel(
        body,
        out_type=jax.ShapeDtypeStruct((C * N, CH), jnp.float32),
        mesh=plsc.VectorSubcoreMesh(**_MESH),
        compiler_params=pltpu.CompilerParams(use_tc_tiling_on_sc=False),
        scratch_types=[
            pltpu.VMEM((NBA, BA), jnp.int32),
            pltpu.VMEM((NBA, BA), jnp.int32),
            pltpu.VMEM((4, BA, CH), jnp.float32),
            pltpu.VMEM((4, BA, CH), jnp.float32),
            pltpu.VMEM((ZR, CH), jnp.float32),
            pltpu.VMEM_SHARED((N + SINK, CH), jnp.float32),
            pltpu.SemaphoreType.DMA,
            pltpu.SemaphoreType.DMA,
        ],
    )


BR = 400
GRID = N // BR


def _dinv_from(deg_ref):
    deg = deg_ref[0, :, 0] + deg_ref[1, :, 0] + 1.0
    return lax.rsqrt(deg)[:, None]


def _tc1_body(x_ref, w_ref, deg_ref, u_ref):
    dinv = _dinv_from(deg_ref)
    u = jnp.dot(x_ref[...], w_ref[...],
                preferred_element_type=jnp.float32) * dinv
    for k in range(C1):
        u_ref[k] = u[:, k * CH:(k + 1) * CH]


_tc1_call = pl.pallas_call(
    _tc1_body,
    grid=(GRID,),
    in_specs=[
        pl.BlockSpec((BR, D_IN), lambda i: (i, 0)),
        pl.BlockSpec((D_IN, D_HID), lambda i: (0, 0)),
        pl.BlockSpec((2, BR, 16), lambda i: (0, i, 0)),
    ],
    out_specs=pl.BlockSpec((C1, BR, CH), lambda i: (0, i, 0)),
    out_shape=jax.ShapeDtypeStruct((C1, N, CH), jnp.float32),
)


def _tc2_body(agg_ref, u_ref, deg_ref, b1_ref, w2_ref, v_ref):
    dinv = _dinv_from(deg_ref)
    pre = jnp.concatenate(
        [agg_ref[k] + u_ref[k] for k in range(C1)], axis=1)
    h = jnp.maximum(pre * dinv + b1_ref[0], 0.0)
    v = jnp.dot(h, w2_ref[...], preferred_element_type=jnp.float32) * dinv
    for k in range(C2):
        v_ref[k] = v[:, k * CH:(k + 1) * CH]


_tc2_call = pl.pallas_call(
    _tc2_body,
    grid=(GRID,),
    in_specs=[
        pl.BlockSpec((C1, BR, CH), lambda i: (0, i, 0)),
        pl.BlockSpec((C1, BR, CH), lambda i: (0, i, 0)),
        pl.BlockSpec((2, BR, 16), lambda i: (0, i, 0)),
        pl.BlockSpec((1, D_HID), lambda i: (0, 0)),
        pl.BlockSpec((D_HID, D_OUT), lambda i: (0, 0)),
    ],
    out_specs=pl.BlockSpec((C2, BR, CH), lambda i: (0, i, 0)),
    out_shape=jax.ShapeDtypeStruct((C2, N, CH), jnp.float32),
)


def _tc3_body(agg_ref, v_ref, deg_ref, b2_ref, o_ref):
    dinv = _dinv_from(deg_ref)
    for k in range(C2):
        o_ref[:, k * CH:(k + 1) * CH] = jnp.maximum(
            (agg_ref[k] + v_ref[k]) * dinv + b2_ref[0, k * CH:(k + 1) * CH],
            0.0)


_tc3_call = pl.pallas_call(
    _tc3_body,
    grid=(GRID,),
    in_specs=[
        pl.BlockSpec((C2, BR, CH), lambda i: (0, i, 0)),
        pl.BlockSpec((C2, BR, CH), lambda i: (0, i, 0)),
        pl.BlockSpec((2, BR, 16), lambda i: (0, i, 0)),
        pl.BlockSpec((1, D_OUT), lambda i: (0, 0)),
    ],
    out_specs=pl.BlockSpec((BR, D_OUT), lambda i: (i, 0)),
    out_shape=jax.ShapeDtypeStruct((N, D_OUT), jnp.float32),
)


def _prep_indices(edge_index):
    row = edge_index[0].astype(jnp.int32)
    col = edge_index[1].astype(jnp.int32)

    row_t = row.reshape(NS, EPT_A)
    col_t = col.reshape(NS, EPT_A)
    prange = jnp.arange(PAD_A, dtype=jnp.int32)
    pad_row = (prange * 997) % N
    pad_col = N + (prange % SINK)
    row_tp = jnp.concatenate(
        [row_t, jnp.broadcast_to(pad_row, (NS, PAD_A))], axis=1)
    col_tp = jnp.concatenate(
        [col_t, jnp.broadcast_to(pad_col, (NS, PAD_A))], axis=1)
    col_a = col_tp.reshape(NS * NBA, BA)
    offs = (jnp.arange(C1, dtype=jnp.int32) * N)[:, None, None]
    rowg1 = (row_tp[None] + offs).reshape(C1 * NS * NBA, BA)
    rowg2 = (row_tp[None] + offs[:C2]).reshape(C2 * NS * NBA, BA)

    col_d = col.reshape(NC * NS, EPT_D)
    drange = jnp.arange(PAD_D, dtype=jnp.int32)
    dpad = N + (drange % SINK)
    col_dp = jnp.concatenate(
        [col_d, jnp.broadcast_to(dpad, (NC * NS, PAD_D))], axis=1)
    col_deg = col_dp.reshape(NC * NS * NBD, BA)
    return rowg1, rowg2, col_a, col_deg


def kernel(x, edge_index, W1, b1, W2, b2):
    rowg1, rowg2, col_a, col_deg = _prep_indices(edge_index)

    deg2 = _deg_call()(col_deg).reshape(NC, N, 16)
    u = _tc1_call(x, W1, deg2)
    agg1 = _make_agg(C1)(u.reshape(C1 * N, CH), rowg1,
                         col_a).reshape(C1, N, CH)
    v = _tc2_call(agg1, u, deg2, b1.reshape(1, D_HID), W2)
    agg2 = _make_agg(C2)(v.reshape(C2 * N, CH), rowg2,
                         col_a).reshape(C2, N, CH)
    return _tc3_call(agg2, v, deg2, b2.reshape(1, D_OUT))
